# baseline (speedup 1.0000x reference)
import functools

import jax
import jax.numpy as jnp
from jax import lax
from jax.experimental import pallas as pl
from jax.experimental.pallas import tpu as pltpu
from jax.experimental.pallas import tpu_sc as plsc

N_NODES = 10000
N_EDGES = 320000
FEAT = 128

NS = 16
CHUNK = 128
T_CHUNKS = 2560
CPT = 160
P = 40
E_PAD = T_CHUNKS * CHUNK
N_PAD = 10240
ROWS_PER_TILE = N_PAD // NS


def _edge_loop(x_hbm, agg_sh, src_v, dst_v, rows0, rows1, sem0, sem1, nchunks):
    pltpu.make_async_copy(x_hbm.at[src_v.at[0]], rows0, sem0).start()

    def _step(i, carry):
        c0 = 2 * i
        c1 = c0 + 1
        pltpu.make_async_copy(x_hbm.at[src_v.at[c0]], rows0, sem0).wait()
        pltpu.make_async_copy(x_hbm.at[src_v.at[c1]], rows1, sem1).start()
        pltpu.sync_copy(rows0, agg_sh.at[dst_v.at[c0]], add=True)

        pltpu.make_async_copy(x_hbm.at[src_v.at[c1]], rows1, sem1).wait()

        @pl.when(c1 + 1 < nchunks)
        def _():
            pltpu.make_async_copy(x_hbm.at[src_v.at[c1 + 1]], rows0, sem0).start()

        pltpu.sync_copy(rows1, agg_sh.at[dst_v.at[c1]], add=True)
        return carry

    lax.fori_loop(0, nchunks // 2, _step, 0)


def _sc_agg_kernel(x_hbm, src_hbm, dst_hbm, out_hbm,
                   agg_sh, src_v, dst_v, rows0, rows1, zbuf, sem0, sem1):
    cid = lax.axis_index("c")
    sid = lax.axis_index("s")

    @pl.when(cid == 0)
    def _work():
        for r in range(8):
            for j in range(8):
                zbuf[r, pl.ds(j * 16, 16)] = jnp.zeros((16,), jnp.float32)

        def _zero(i, carry):
            pltpu.sync_copy(zbuf,
                            agg_sh.at[pl.ds(sid * ROWS_PER_TILE + i * 8, 8)])
            return carry

        with jax.named_scope("agg_zero"):
            lax.fori_loop(0, ROWS_PER_TILE // 8, _zero, 0)
            plsc.subcore_barrier()

        def _phase(p, carry):
            base = sid * CPT + p * P
            pltpu.sync_copy(src_hbm.at[pl.ds(base, P)], src_v)
            pltpu.sync_copy(dst_hbm.at[pl.ds(base, P)], dst_v)
            _edge_loop(x_hbm, agg_sh, src_v, dst_v, rows0, rows1,
                       sem0, sem1, P)
            return carry

        with jax.named_scope("agg_edges"):
            lax.fori_loop(0, CPT // P, _phase, 0)
            plsc.subcore_barrier()

        with jax.named_scope("agg_readout"):
            for k in range(ROWS_PER_TILE // CHUNK):
                base = sid * ROWS_PER_TILE + k * CHUNK
                pltpu.sync_copy(agg_sh.at[pl.ds(base, CHUNK)], rows0)
                pltpu.sync_copy(rows0, out_hbm.at[pl.ds(base, CHUNK)])


def _sc_aggregate(x, src_g, dst_g):
    mesh = plsc.VectorSubcoreMesh(core_axis_name="c", subcore_axis_name="s")
    k = pl.kernel(
        _sc_agg_kernel,
        out_type=jax.ShapeDtypeStruct((N_PAD, FEAT), jnp.float32),
        mesh=mesh,
        scratch_types=[
            pltpu.VMEM_SHARED((N_PAD, FEAT), jnp.float32),
            pltpu.VMEM((P, CHUNK), jnp.int32),
            pltpu.VMEM((P, CHUNK), jnp.int32),
            pltpu.VMEM((CHUNK, FEAT), jnp.float32),
            pltpu.VMEM((CHUNK, FEAT), jnp.float32),
            pltpu.VMEM((8, FEAT), jnp.float32),
            pltpu.SemaphoreType.DMA,
            pltpu.SemaphoreType.DMA,
        ],
    )
    return k(x, src_g, dst_g)


def _mlp_body(x_ref, a_ref, w1_ref, b1_ref, w2_ref, b2_ref, o_ref,
              *, final_relu):
    h = x_ref[...] + a_ref[...]
    t = jnp.dot(h, w1_ref[...], preferred_element_type=jnp.float32) + b1_ref[...]
    t = jnp.maximum(t, 0.0)
    u = jnp.dot(t, w2_ref[...], preferred_element_type=jnp.float32) + b2_ref[...]
    if final_relu:
        u = jnp.maximum(u, 0.0)
    o_ref[...] = u


def _tc_mlp(x, a, W1, b1, W2, b2, final_relu):
    n = x.shape[0]
    blk = 2000
    grid = n // blk
    b1r = b1.reshape(1, FEAT)
    b2r = b2.reshape(1, FEAT)
    row_spec = pl.BlockSpec((blk, FEAT), lambda i: (i, 0))
    full_w = pl.BlockSpec((FEAT, FEAT), lambda i: (0, 0))
    full_b = pl.BlockSpec((1, FEAT), lambda i: (0, 0))
    return pl.pallas_call(
        functools.partial(_mlp_body, final_relu=final_relu),
        grid=(grid,),
        in_specs=[row_spec, row_spec, full_w, full_b, full_w, full_b],
        out_specs=row_spec,
        out_shape=jax.ShapeDtypeStruct((n, FEAT), jnp.float32),
    )(x, a, W1, b1r, W2, b2r)


def kernel(x, edge_index, W1a, b1a, W2a, b2a, W1b, b1b, W2b, b2b):
    src = edge_index[0].astype(jnp.int32)
    dst = edge_index[1].astype(jnp.int32)
    pad = E_PAD - N_EDGES
    src_g = jnp.concatenate([src, jnp.zeros((pad,), jnp.int32)]).reshape(
        T_CHUNKS, CHUNK)
    dst_g = jnp.concatenate([dst, jnp.full((pad,), N_PAD - 1, jnp.int32)]
                            ).reshape(T_CHUNKS, CHUNK)

    agg_x = _sc_aggregate(x, src_g, dst_g)
    h = _tc_mlp(x, agg_x[:N_NODES], W1a, b1a, W2a, b2a, final_relu=True)
    agg_h = _sc_aggregate(h, src_g, dst_g)
    out = _tc_mlp(h, agg_h[:N_NODES], W1b, b1b, W2b, b2b, final_relu=False)
    return out


# speedup vs baseline: 1.2163x; 1.2163x over previous
import functools

import jax
import jax.numpy as jnp
from jax import lax
from jax.experimental import pallas as pl
from jax.experimental.pallas import tpu as pltpu
from jax.experimental.pallas import tpu_sc as plsc

N_NODES = 10000
N_EDGES = 320000
FEAT = 128

NS = 16
CHUNK = 128
T_CHUNKS = 2560
CH0 = 128
CH1 = 160 - CH0
P = 32
E_PAD = T_CHUNKS * CHUNK
N_PAD = 10240
ROWS_PER_TILE = N_PAD // NS


def _edge_loop(x_hbm, agg_sh, src_v, dst_v, rows0, rows1, sem0, sem1, nchunks):
    pltpu.make_async_copy(x_hbm.at[src_v.at[0]], rows0, sem0).start()

    def _step(i, carry):
        c0 = 2 * i
        c1 = c0 + 1
        pltpu.make_async_copy(x_hbm.at[src_v.at[c0]], rows0, sem0).wait()
        pltpu.make_async_copy(x_hbm.at[src_v.at[c1]], rows1, sem1).start()
        pltpu.sync_copy(rows0, agg_sh.at[dst_v.at[c0]], add=True)

        pltpu.make_async_copy(x_hbm.at[src_v.at[c1]], rows1, sem1).wait()

        @pl.when(c1 + 1 < nchunks)
        def _():
            pltpu.make_async_copy(x_hbm.at[src_v.at[c1 + 1]], rows0, sem0).start()

        pltpu.sync_copy(rows1, agg_sh.at[dst_v.at[c1]], add=True)
        return carry

    lax.fori_loop(0, nchunks // 2, _step, 0)


def _sc_agg_kernel(x_hbm, src_hbm, dst_hbm, out_hbm,
                   agg_sh, src_v, dst_v, rows0, rows1, zbuf, sem0, sem1):
    cid = lax.axis_index("c")
    sid = lax.axis_index("s")

    for r in range(8):
        for j in range(8):
            zbuf[r, pl.ds(j * 16, 16)] = jnp.zeros((16,), jnp.float32)

    def _zero(i, carry):
        pltpu.sync_copy(zbuf, agg_sh.at[pl.ds(sid * ROWS_PER_TILE + i * 8, 8)])
        return carry

    with jax.named_scope("agg_zero"):
        lax.fori_loop(0, ROWS_PER_TILE // 8, _zero, 0)
        plsc.subcore_barrier()

    start = jnp.where(cid == 0, sid * CH0, NS * CH0 + sid * CH1)
    nph = jnp.where(cid == 0, CH0 // P, CH1 // P)

    def _phase(p, carry):
        with jax.named_scope("agg_phase"):
            base = start + p * P
            pltpu.sync_copy(src_hbm.at[pl.ds(base, P)], src_v)
            pltpu.sync_copy(dst_hbm.at[pl.ds(base, P)], dst_v)
            _edge_loop(x_hbm, agg_sh, src_v, dst_v, rows0, rows1,
                       sem0, sem1, P)
        return carry

    with jax.named_scope("agg_edges"):
        lax.fori_loop(0, nph, _phase, 0)
        plsc.subcore_barrier()

    with jax.named_scope("agg_readout"):
        for k in range(ROWS_PER_TILE // CHUNK):
            base = sid * ROWS_PER_TILE + k * CHUNK
            pltpu.sync_copy(agg_sh.at[pl.ds(base, CHUNK)], rows0)
            pltpu.sync_copy(rows0, out_hbm.at[cid, pl.ds(base, CHUNK)])


NC = 2


def _sc_aggregate(x, src_g, dst_g):
    mesh = plsc.VectorSubcoreMesh(core_axis_name="c", subcore_axis_name="s")
    k = pl.kernel(
        _sc_agg_kernel,
        out_type=jax.ShapeDtypeStruct((NC, N_PAD, FEAT), jnp.float32),
        mesh=mesh,
        scratch_types=[
            pltpu.VMEM_SHARED((N_PAD, FEAT), jnp.float32),
            pltpu.VMEM((P, CHUNK), jnp.int32),
            pltpu.VMEM((P, CHUNK), jnp.int32),
            pltpu.VMEM((CHUNK, FEAT), jnp.float32),
            pltpu.VMEM((CHUNK, FEAT), jnp.float32),
            pltpu.VMEM((8, FEAT), jnp.float32),
            pltpu.SemaphoreType.DMA,
            pltpu.SemaphoreType.DMA,
        ],
    )
    return k(x, src_g, dst_g)


def _mlp_body(x_ref, a0_ref, a1_ref, w1_ref, b1_ref, w2_ref, b2_ref, o_ref,
              *, final_relu):
    h = x_ref[...] + a0_ref[0] + a1_ref[0]
    t = jnp.dot(h, w1_ref[...], preferred_element_type=jnp.float32) + b1_ref[...]
    t = jnp.maximum(t, 0.0)
    u = jnp.dot(t, w2_ref[...], preferred_element_type=jnp.float32) + b2_ref[...]
    if final_relu:
        u = jnp.maximum(u, 0.0)
    o_ref[...] = u


def _tc_mlp(x, agg, W1, b1, W2, b2, final_relu):
    n = x.shape[0]
    blk = 2000
    grid = n // blk
    b1r = b1.reshape(1, FEAT)
    b2r = b2.reshape(1, FEAT)
    row_spec = pl.BlockSpec((blk, FEAT), lambda i: (i, 0))
    agg0_spec = pl.BlockSpec((1, blk, FEAT), lambda i: (0, i, 0))
    agg1_spec = pl.BlockSpec((1, blk, FEAT), lambda i: (1, i, 0))
    full_w = pl.BlockSpec((FEAT, FEAT), lambda i: (0, 0))
    full_b = pl.BlockSpec((1, FEAT), lambda i: (0, 0))
    return pl.pallas_call(
        functools.partial(_mlp_body, final_relu=final_relu),
        grid=(grid,),
        in_specs=[row_spec, agg0_spec, agg1_spec, full_w, full_b, full_w,
                  full_b],
        out_specs=row_spec,
        out_shape=jax.ShapeDtypeStruct((n, FEAT), jnp.float32),
    )(x, agg, agg, W1, b1r, W2, b2r)


def kernel(x, edge_index, W1a, b1a, W2a, b2a, W1b, b1b, W2b, b2b):
    src = edge_index[0].astype(jnp.int32)
    dst = edge_index[1].astype(jnp.int32)
    pad = E_PAD - N_EDGES
    src_g = jnp.concatenate([src, jnp.zeros((pad,), jnp.int32)]).reshape(
        T_CHUNKS, CHUNK)
    dst_g = jnp.concatenate([dst, jnp.full((pad,), N_PAD - 1, jnp.int32)]
                            ).reshape(T_CHUNKS, CHUNK)

    agg_x = _sc_aggregate(x, src_g, dst_g)
    h = _tc_mlp(x, agg_x, W1a, b1a, W2a, b2a, final_relu=True)
    agg_h = _sc_aggregate(h, src_g, dst_g)
    out = _tc_mlp(h, agg_h, W1b, b1b, W2b, b2b, final_relu=False)
    return out


# speedup vs baseline: 3.5347x; 2.9062x over previous
import functools

import jax
import jax.numpy as jnp
from jax import lax
from jax.experimental import pallas as pl
from jax.experimental.pallas import tpu as pltpu
from jax.experimental.pallas import tpu_sc as plsc

N_NODES = 10000
N_EDGES = 320000
FEAT = 128

NS = 16
CHUNK = 128
T_CHUNKS = 2560
CH0 = 80
CH1 = 160 - CH0
P = 40
E_PAD = T_CHUNKS * CHUNK
N_PAD = 10240
ROWS_PER_TILE = N_PAD // NS


def _edge_loop(x_hbm, agg_sh, src_v, dst_v, rows0, rows1, sem0, sem1, nchunks):
    pltpu.make_async_copy(x_hbm.at[src_v.at[0]], rows0, sem0).start()

    def _step(i, carry):
        c0 = 2 * i
        c1 = c0 + 1
        pltpu.make_async_copy(x_hbm.at[src_v.at[c0]], rows0, sem0).wait()
        pltpu.make_async_copy(x_hbm.at[src_v.at[c1]], rows1, sem1).start()
        pltpu.sync_copy(rows0, agg_sh.at[dst_v.at[c0]], add=True)

        pltpu.make_async_copy(x_hbm.at[src_v.at[c1]], rows1, sem1).wait()

        @pl.when(c1 + 1 < nchunks)
        def _():
            pltpu.make_async_copy(x_hbm.at[src_v.at[c1 + 1]], rows0, sem0).start()

        pltpu.sync_copy(rows1, agg_sh.at[dst_v.at[c1]], add=True)
        return carry

    lax.fori_loop(0, nchunks // 2, _step, 0)


def _sc_agg_kernel(x_hbm, src_hbm, dst_hbm, out_hbm,
                   agg_sh, src_v, dst_v, rows0, rows1, zbuf, sem0, sem1):
    cid = lax.axis_index("c")
    sid = lax.axis_index("s")

    for r in range(8):
        for j in range(8):
            zbuf[r, pl.ds(j * 16, 16)] = jnp.zeros((16,), jnp.float32)

    def _zero(i, carry):
        pltpu.sync_copy(zbuf, agg_sh.at[pl.ds(sid * ROWS_PER_TILE + i * 8, 8)])
        return carry

    with jax.named_scope("agg_zero"):
        lax.fori_loop(0, ROWS_PER_TILE // 8, _zero, 0)
        plsc.subcore_barrier()

    start = jnp.where(cid == 0, sid * CH0, NS * CH0 + sid * CH1)
    nph = jnp.where(cid == 0, CH0 // P, CH1 // P)

    def _phase(p, carry):
        with jax.named_scope("agg_phase"):
            base = start + p * P
            pltpu.sync_copy(src_hbm.at[pl.ds(base, P)], src_v)
            pltpu.sync_copy(dst_hbm.at[pl.ds(base, P)], dst_v)
            _edge_loop(x_hbm, agg_sh, src_v, dst_v, rows0, rows1,
                       sem0, sem1, P)
        return carry

    with jax.named_scope("agg_edges"):
        lax.fori_loop(0, nph, _phase, 0)
        plsc.subcore_barrier()

    with jax.named_scope("agg_readout"):
        for k in range(ROWS_PER_TILE // CHUNK):
            base = sid * ROWS_PER_TILE + k * CHUNK
            pltpu.sync_copy(agg_sh.at[pl.ds(base, CHUNK)], rows0)
            pltpu.sync_copy(rows0, out_hbm.at[cid, pl.ds(base, CHUNK)])


NC = 2


def _sc_aggregate(x, src_g, dst_g):
    mesh = plsc.VectorSubcoreMesh(core_axis_name="c", subcore_axis_name="s")
    k = pl.kernel(
        _sc_agg_kernel,
        out_type=jax.ShapeDtypeStruct((NC, N_PAD, FEAT), jnp.float32),
        mesh=mesh,
        scratch_types=[
            pltpu.VMEM_SHARED((N_PAD, FEAT), jnp.float32),
            pltpu.VMEM((P, CHUNK), jnp.int32),
            pltpu.VMEM((P, CHUNK), jnp.int32),
            pltpu.VMEM((CHUNK, FEAT), jnp.float32),
            pltpu.VMEM((CHUNK, FEAT), jnp.float32),
            pltpu.VMEM((8, FEAT), jnp.float32),
            pltpu.SemaphoreType.DMA,
            pltpu.SemaphoreType.DMA,
        ],
    )
    return k(x, src_g, dst_g)


def _mlp_body(x_ref, a0_ref, a1_ref, w1_ref, b1_ref, w2_ref, b2_ref, o_ref,
              *, final_relu):
    h = x_ref[...] + a0_ref[0] + a1_ref[0]
    t = jnp.dot(h, w1_ref[...], preferred_element_type=jnp.float32) + b1_ref[...]
    t = jnp.maximum(t, 0.0)
    u = jnp.dot(t, w2_ref[...], preferred_element_type=jnp.float32) + b2_ref[...]
    if final_relu:
        u = jnp.maximum(u, 0.0)
    o_ref[...] = u


def _tc_mlp(x, agg, W1, b1, W2, b2, final_relu):
    n = x.shape[0]
    blk = 2000
    grid = n // blk
    b1r = b1.reshape(1, FEAT)
    b2r = b2.reshape(1, FEAT)
    row_spec = pl.BlockSpec((blk, FEAT), lambda i: (i, 0))
    agg0_spec = pl.BlockSpec((1, blk, FEAT), lambda i: (0, i, 0))
    agg1_spec = pl.BlockSpec((1, blk, FEAT), lambda i: (1, i, 0))
    full_w = pl.BlockSpec((FEAT, FEAT), lambda i: (0, 0))
    full_b = pl.BlockSpec((1, FEAT), lambda i: (0, 0))
    return pl.pallas_call(
        functools.partial(_mlp_body, final_relu=final_relu),
        grid=(grid,),
        in_specs=[row_spec, agg0_spec, agg1_spec, full_w, full_b, full_w,
                  full_b],
        out_specs=row_spec,
        out_shape=jax.ShapeDtypeStruct((n, FEAT), jnp.float32),
    )(x, agg, agg, W1, b1r, W2, b2r)


def kernel(x, edge_index, W1a, b1a, W2a, b2a, W1b, b1b, W2b, b2b):
    src = edge_index[0].astype(jnp.int32)
    dst = edge_index[1].astype(jnp.int32)
    pad = E_PAD - N_EDGES
    pad_i = jnp.arange(pad, dtype=jnp.int32)
    src_g = jnp.concatenate([src, pad_i % N_NODES]).reshape(T_CHUNKS, CHUNK)
    dst_g = jnp.concatenate([dst, N_NODES + pad_i % (N_PAD - N_NODES)]
                            ).reshape(T_CHUNKS, CHUNK)

    agg_x = _sc_aggregate(x, src_g, dst_g)
    h = _tc_mlp(x, agg_x, W1a, b1a, W2a, b2a, final_relu=True)
    agg_h = _sc_aggregate(h, src_g, dst_g)
    out = _tc_mlp(h, agg_h, W1b, b1b, W2b, b2b, final_relu=False)
    return out


# speedup vs baseline: 3.5546x; 1.0056x over previous
import functools

import jax
import jax.numpy as jnp
import numpy as np
from jax import lax
from jax.experimental import pallas as pl
from jax.experimental.pallas import tpu as pltpu
from jax.experimental.pallas import tpu_sc as plsc

N_NODES = 10000
N_EDGES = 320000
FEAT = 128

NS = 16
NC = 2
CHUNK = 128
T_CHUNKS = 2560
CPT = 80
P = 40
N_PAD = 10240
ROWS_PER_TILE = N_PAD // NS

_N_PAD_EDGES = T_CHUNKS * CHUNK - N_EDGES
_PAD_SRC = np.arange(_N_PAD_EDGES, dtype=np.int32) % N_NODES
_PAD_DST = N_NODES + np.arange(_N_PAD_EDGES, dtype=np.int32) % (N_PAD - N_NODES)


def _edge_loop(x_hbm, agg_sh, src_v, dst_v, rows0, rows1, sem0, sem1, nchunks):
    pltpu.make_async_copy(x_hbm.at[src_v.at[0]], rows0, sem0).start()

    def _step(i, carry):
        c0 = 2 * i
        c1 = c0 + 1
        pltpu.make_async_copy(x_hbm.at[src_v.at[c0]], rows0, sem0).wait()
        pltpu.make_async_copy(x_hbm.at[src_v.at[c1]], rows1, sem1).start()
        pltpu.sync_copy(rows0, agg_sh.at[dst_v.at[c0]], add=True)

        pltpu.make_async_copy(x_hbm.at[src_v.at[c1]], rows1, sem1).wait()

        @pl.when(c1 + 1 < nchunks)
        def _():
            pltpu.make_async_copy(x_hbm.at[src_v.at[c1 + 1]], rows0, sem0).start()

        pltpu.sync_copy(rows1, agg_sh.at[dst_v.at[c1]], add=True)
        return carry

    lax.fori_loop(0, nchunks // 2, _step, 0)


def _sc_agg_kernel(x_hbm, src_hbm, dst_hbm, out_hbm,
                   agg_sh, src_v, dst_v, rows0, rows1, zbuf, sem0, sem1):
    cid = lax.axis_index("c")
    sid = lax.axis_index("s")

    for r in range(8):
        for j in range(8):
            zbuf[r, pl.ds(j * 16, 16)] = jnp.zeros((16,), jnp.float32)

    def _zero(i, carry):
        pltpu.sync_copy(zbuf, agg_sh.at[pl.ds(sid * ROWS_PER_TILE + i * 8, 8)])
        return carry

    with jax.named_scope("agg_zero"):
        lax.fori_loop(0, ROWS_PER_TILE // 8, _zero, 0)
        plsc.subcore_barrier()

    wid = sid * NC + cid
    start = wid * CPT
    nph = CPT // P

    def _phase(p, carry):
        with jax.named_scope("agg_phase"):
            base = start + p * P
            pltpu.sync_copy(src_hbm.at[pl.ds(base, P)], src_v)
            pltpu.sync_copy(dst_hbm.at[pl.ds(base, P)], dst_v)
            _edge_loop(x_hbm, agg_sh, src_v, dst_v, rows0, rows1,
                       sem0, sem1, P)
        return carry

    with jax.named_scope("agg_edges"):
        lax.fori_loop(0, nph, _phase, 0)
        plsc.subcore_barrier()

    with jax.named_scope("agg_readout"):
        for k in range(ROWS_PER_TILE // CHUNK):
            base = sid * ROWS_PER_TILE + k * CHUNK
            pltpu.sync_copy(agg_sh.at[pl.ds(base, CHUNK)], rows0)
            pltpu.sync_copy(rows0, out_hbm.at[cid, pl.ds(base, CHUNK)])


def _sc_aggregate(x, src_g, dst_g):
    mesh = plsc.VectorSubcoreMesh(core_axis_name="c", subcore_axis_name="s")
    k = pl.kernel(
        _sc_agg_kernel,
        out_type=jax.ShapeDtypeStruct((NC, N_PAD, FEAT), jnp.float32),
        mesh=mesh,
        scratch_types=[
            pltpu.VMEM_SHARED((N_PAD, FEAT), jnp.float32),
            pltpu.VMEM((P, CHUNK), jnp.int32),
            pltpu.VMEM((P, CHUNK), jnp.int32),
            pltpu.VMEM((CHUNK, FEAT), jnp.float32),
            pltpu.VMEM((CHUNK, FEAT), jnp.float32),
            pltpu.VMEM((8, FEAT), jnp.float32),
            pltpu.SemaphoreType.DMA,
            pltpu.SemaphoreType.DMA,
        ],
    )
    return k(x, src_g, dst_g)


def _mlp_body(x_ref, a0_ref, a1_ref, w1_ref, b1_ref, w2_ref, b2_ref, o_ref,
              *, final_relu):
    h = x_ref[...] + a0_ref[0] + a1_ref[0]
    t = jnp.dot(h, w1_ref[...], preferred_element_type=jnp.float32) + b1_ref[...]
    t = jnp.maximum(t, 0.0)
    u = jnp.dot(t, w2_ref[...], preferred_element_type=jnp.float32) + b2_ref[...]
    if final_relu:
        u = jnp.maximum(u, 0.0)
    o_ref[...] = u


def _tc_mlp(x, agg, W1, b1, W2, b2, final_relu):
    n = x.shape[0]
    blk = 2000
    grid = n // blk
    b1r = b1.reshape(1, FEAT)
    b2r = b2.reshape(1, FEAT)
    row_spec = pl.BlockSpec((blk, FEAT), lambda i: (i, 0))
    agg0_spec = pl.BlockSpec((1, blk, FEAT), lambda i: (0, i, 0))
    agg1_spec = pl.BlockSpec((1, blk, FEAT), lambda i: (1, i, 0))
    full_w = pl.BlockSpec((FEAT, FEAT), lambda i: (0, 0))
    full_b = pl.BlockSpec((1, FEAT), lambda i: (0, 0))
    return pl.pallas_call(
        functools.partial(_mlp_body, final_relu=final_relu),
        grid=(grid,),
        in_specs=[row_spec, agg0_spec, agg1_spec, full_w, full_b, full_w,
                  full_b],
        out_specs=row_spec,
        out_shape=jax.ShapeDtypeStruct((n, FEAT), jnp.float32),
    )(x, agg, agg, W1, b1r, W2, b2r)


def kernel(x, edge_index, W1a, b1a, W2a, b2a, W1b, b1b, W2b, b2b):
    src_g = jnp.concatenate([edge_index[0].astype(jnp.int32), _PAD_SRC]
                            ).reshape(T_CHUNKS, CHUNK)
    dst_g = jnp.concatenate([edge_index[1].astype(jnp.int32), _PAD_DST]
                            ).reshape(T_CHUNKS, CHUNK)

    agg_x = _sc_aggregate(x, src_g, dst_g)
    h = _tc_mlp(x, agg_x, W1a, b1a, W2a, b2a, final_relu=True)
    agg_h = _sc_aggregate(h, src_g, dst_g)
    out = _tc_mlp(h, agg_h, W1b, b1b, W2b, b2b, final_relu=False)
    return out
